# baseline (speedup 1.0000x reference)
import functools

import jax
import jax.numpy as jnp
from jax import lax
from jax.experimental import pallas as pl
from jax.experimental.pallas import tpu as pltpu
from jax.experimental.pallas import tpu_sc as plsc

NC = 2
NS = 16
NW = NC * NS
CHUNK = 2


def _gather_body(n_chunks, b_per_w, table_hbm, idx_hbm, out_hbm,
                 idx_v, buf0, buf1, g0, g1, s0, s1):
    wid = lax.axis_index("s") * NC + lax.axis_index("c")
    base = wid * b_per_w
    bufs = (buf0, buf1)
    gsems = (g0, g1)
    ssems = (s0, s1)

    pltpu.sync_copy(idx_hbm.at[wid], idx_v)

    pltpu.async_copy(table_hbm.at[idx_v.at[0]], buf0, g0)
    pltpu.async_copy(table_hbm.at[idx_v.at[1]], buf1, g1)

    def step(g, b, start_next):
        pltpu.make_async_copy(table_hbm.at[pl.ds(0, CHUNK)], bufs[b],
                              gsems[b]).wait()
        pltpu.async_copy(bufs[b], out_hbm.at[pl.ds(base + g * CHUNK, CHUNK)],
                         ssems[b])
        pltpu.make_async_copy(table_hbm.at[pl.ds(0, CHUNK)], bufs[b],
                              ssems[b]).wait()
        if start_next:
            pltpu.async_copy(table_hbm.at[idx_v.at[g + 2]], bufs[b], gsems[b])

    @pl.loop(0, n_chunks - 2, step=2)
    def _(gp):
        for b in range(2):
            step(gp + b, b, True)

    for g in (n_chunks - 2, n_chunks - 1):
        step(g, g % 2, False)


def kernel(prefix, embedding_table):
    bsz, toks = prefix.shape
    vocab, dim = embedding_table.shape
    n_lookups = bsz * toks
    b_per_w = n_lookups // NW
    n_chunks = b_per_w // CHUNK

    idx = jnp.asarray(prefix, jnp.int32).reshape(NW, n_chunks, CHUNK)

    mesh = plsc.VectorSubcoreMesh(core_axis_name="c", subcore_axis_name="s")
    sc_gather = pl.kernel(
        functools.partial(_gather_body, n_chunks, b_per_w),
        out_type=jax.ShapeDtypeStruct((n_lookups, dim), jnp.float32),
        mesh=mesh,
        scratch_types=[
            pltpu.VMEM((n_chunks, CHUNK), jnp.int32),
            pltpu.VMEM((CHUNK, dim), jnp.float32),
            pltpu.VMEM((CHUNK, dim), jnp.float32),
            pltpu.SemaphoreType.DMA,
            pltpu.SemaphoreType.DMA,
            pltpu.SemaphoreType.DMA,
            pltpu.SemaphoreType.DMA,
        ],
    )
    out = sc_gather(embedding_table, idx)
    return out.reshape(bsz, toks, dim)


# speedup vs baseline: 1.7246x; 1.7246x over previous
import functools

import jax
import jax.numpy as jnp
from jax import lax
from jax.experimental import pallas as pl
from jax.experimental.pallas import tpu as pltpu
from jax.experimental.pallas import tpu_sc as plsc

NC = 2
NS = 16
NW = NC * NS
CHUNK = 2


def _gather_body(n_chunks, b_per_w, toks, table_hbm, idx_hbm, out_hbm,
                 idx_v, buf0, buf1, g0, g1, s0, s1):
    wid = lax.axis_index("s") * NC + lax.axis_index("c")
    base = wid * b_per_w
    bufs = (buf0, buf1)
    gsems = (g0, g1)
    ssems = (s0, s1)

    pltpu.sync_copy(idx_hbm.at[wid], idx_v)

    pltpu.async_copy(table_hbm.at[idx_v.at[0]], buf0, g0)
    pltpu.async_copy(table_hbm.at[idx_v.at[1]], buf1, g1)

    def step(g, b, start_next):
        pltpu.make_async_copy(table_hbm.at[pl.ds(0, CHUNK)], bufs[b],
                              gsems[b]).wait()
        row = base + g * CHUNK
        pltpu.async_copy(bufs[b],
                         out_hbm.at[row // toks, pl.ds(row % toks, CHUNK)],
                         ssems[b])
        pltpu.make_async_copy(table_hbm.at[pl.ds(0, CHUNK)], bufs[b],
                              ssems[b]).wait()
        if start_next:
            pltpu.async_copy(table_hbm.at[idx_v.at[g + 2]], bufs[b], gsems[b])

    @pl.loop(0, n_chunks - 2, step=2)
    def _(gp):
        for b in range(2):
            step(gp + b, b, True)

    for g in (n_chunks - 2, n_chunks - 1):
        step(g, g % 2, False)


def kernel(prefix, embedding_table):
    bsz, toks = prefix.shape
    vocab, dim = embedding_table.shape
    n_lookups = bsz * toks
    b_per_w = n_lookups // NW
    n_chunks = b_per_w // CHUNK

    idx = jnp.asarray(prefix, jnp.int32).reshape(NW, n_chunks, CHUNK)

    mesh = plsc.VectorSubcoreMesh(core_axis_name="c", subcore_axis_name="s")
    sc_gather = pl.kernel(
        functools.partial(_gather_body, n_chunks, b_per_w, toks),
        out_type=jax.ShapeDtypeStruct((bsz, toks, dim), jnp.float32),
        mesh=mesh,
        scratch_types=[
            pltpu.VMEM((n_chunks, CHUNK), jnp.int32),
            pltpu.VMEM((CHUNK, dim), jnp.float32),
            pltpu.VMEM((CHUNK, dim), jnp.float32),
            pltpu.SemaphoreType.DMA,
            pltpu.SemaphoreType.DMA,
            pltpu.SemaphoreType.DMA,
            pltpu.SemaphoreType.DMA,
        ],
    )
    return sc_gather(embedding_table, idx)


# speedup vs baseline: 3.2031x; 1.8572x over previous
import functools

import jax
import jax.numpy as jnp
from jax import lax
from jax.experimental import pallas as pl
from jax.experimental.pallas import tpu as pltpu
from jax.experimental.pallas import tpu_sc as plsc

NC = 2
NS = 16
NW = NC * NS
NBUF = 4
LANES = 16
RUN_PAD = 224


def _extract(ref, j):
    vec = ref[j // LANES, :]
    lane = lax.iota(jnp.int32, LANES)
    return jnp.max(jnp.where(lane == j % LANES, vec, 0))


def _gather_body(n_per_w, bsz, table_hbm, grow_hbm, rstart_hbm, posn_hbm,
                 meta_hbm, out_hbm, grow_v, rstart_v, posn_v, meta_v,
                 buf0, buf1, buf2, buf3, g0, g1, g2, g3, s0, s1, s2, s3):
    wid = lax.axis_index("s") * NC + lax.axis_index("c")
    base = wid * n_per_w
    bufs = (buf0, buf1, buf2, buf3)
    gsems = (g0, g1, g2, g3)
    ssems = (s0, s1, s2, s3)

    def gwait(b):
        pltpu.make_async_copy(table_hbm.at[pl.ds(0, 1)], bufs[b],
                              gsems[b]).wait()

    def swait_n(b, n):
        def one(i, carry):
            pltpu.make_async_copy(table_hbm.at[pl.ds(0, 1)], bufs[b],
                                  ssems[b]).wait()
            return carry
        lax.fori_loop(0, n, one, 0)

    def gather(r, b):
        row = _extract(grow_v, r)
        pltpu.async_copy(table_hbm.at[pl.ds(row, 1)], bufs[b], gsems[b])

    pltpu.sync_copy(grow_hbm.at[wid], grow_v)
    pltpu.sync_copy(rstart_hbm.at[wid], rstart_v)
    pltpu.sync_copy(posn_hbm.at[wid], posn_v)
    pltpu.sync_copy(meta_hbm.at[wid], meta_v)
    nruns = _extract(meta_v, 0)
    nr4 = ((nruns + NBUF - 1) // NBUF) * NBUF

    for r in range(NBUF - 1):
        gather(r, r)

    def step(r, b, carry):
        cnts = list(carry)
        bf = (b + NBUF - 1) % NBUF
        swait_n(bf, cnts[bf])
        gather(r + NBUF - 1, bf)
        gwait(b)
        k0 = _extract(rstart_v, r)
        k1 = _extract(rstart_v, r + 1)

        def one(k, carry2):
            p = base + _extract(posn_v, k)
            pltpu.async_copy(bufs[b],
                             out_hbm.at[p // bsz, pl.ds(p % bsz, 1)],
                             ssems[b])
            return carry2
        lax.fori_loop(k0, k1, one, 0)

        cnts[bf] = 0
        cnts[b] = k1 - k0
        return tuple(cnts)

    def ring_body(rp, carry):
        for b in range(NBUF):
            carry = step(rp + b, b, carry)
        return carry

    carry_out = pl.loop(0, nr4, step=NBUF,
                        init_carry=(jnp.int32(0),) * NBUF)(ring_body)

    for b in range(NBUF - 1):
        gwait(b)
    for b in range(NBUF):
        swait_n(b, carry_out[b])


def kernel(prefix, embedding_table):
    bsz, toks = prefix.shape
    vocab, dim = embedding_table.shape
    n_lookups = bsz * toks
    n_per_w = n_lookups // NW

    seg = jnp.asarray(prefix, jnp.int32).T.reshape(NW, n_per_w)
    offs = (jnp.arange(NW, dtype=jnp.int32) * vocab) // NW
    key = (seg - offs[:, None]) % vocab
    order = jnp.argsort(key, axis=1, stable=True).astype(jnp.int32)
    srt = jnp.take_along_axis(seg, order, axis=1)
    keys = jnp.take_along_axis(key, order, axis=1)
    first = jnp.concatenate(
        [jnp.ones((NW, 1), bool), keys[:, 1:] != keys[:, :-1]], axis=1)
    rid = jnp.cumsum(first, axis=1).astype(jnp.int32) - 1
    nruns = rid[:, -1] + 1

    grow = jnp.zeros((NW, RUN_PAD), jnp.int32)
    grow = grow.at[jnp.arange(NW)[:, None], rid].set(srt)
    run_valid = jnp.arange(RUN_PAD)[None, :] < nruns[:, None]
    grow = jnp.where(run_valid, grow, srt[:, -1:])
    rstart = jax.vmap(
        lambda r, q: jnp.searchsorted(r, q).astype(jnp.int32),
        in_axes=(0, None))(rid, jnp.arange(RUN_PAD, dtype=jnp.int32))

    n_pos = ((n_per_w + LANES - 1) // LANES) * LANES
    posn = jnp.zeros((NW, n_pos), jnp.int32).at[:, :n_per_w].set(order)
    grow = grow.reshape(NW, RUN_PAD // LANES, LANES)
    rstart = rstart.reshape(NW, RUN_PAD // LANES, LANES)
    posn = posn.reshape(NW, n_pos // LANES, LANES)
    meta = jnp.zeros((NW, LANES), jnp.int32).at[:, 0].set(nruns)
    meta = meta.reshape(NW, 1, LANES)

    mesh = plsc.VectorSubcoreMesh(core_axis_name="c", subcore_axis_name="s")
    sc_gather = pl.kernel(
        functools.partial(_gather_body, n_per_w, bsz),
        out_type=jax.ShapeDtypeStruct((toks, bsz, dim), jnp.float32),
        mesh=mesh,
        compiler_params=pltpu.CompilerParams(use_tc_tiling_on_sc=True,
                                             needs_layout_passes=False),
        scratch_types=(
            [pltpu.VMEM((RUN_PAD // LANES, LANES), jnp.int32),
             pltpu.VMEM((RUN_PAD // LANES, LANES), jnp.int32),
             pltpu.VMEM((n_pos // LANES, LANES), jnp.int32),
             pltpu.VMEM((1, LANES), jnp.int32)]
            + [pltpu.VMEM((1, dim), jnp.float32) for _ in range(NBUF)]
            + [pltpu.SemaphoreType.DMA for _ in range(2 * NBUF)]
        ),
    )
    out = sc_gather(embedding_table, grow, rstart, posn, meta)
    return jnp.swapaxes(out, 0, 1)


# speedup vs baseline: 3.5587x; 1.1110x over previous
import functools

import jax
import jax.numpy as jnp
from jax import lax
from jax.experimental import pallas as pl
from jax.experimental.pallas import tpu as pltpu
from jax.experimental.pallas import tpu_sc as plsc

NC = 2
NS = 16
NW = NC * NS
NBUF = 4
LANES = 16
RUN_PAD = 224


def _extract(ref, j):
    vec = ref[j // LANES, :]
    lane = lax.iota(jnp.int32, LANES)
    return jnp.max(jnp.where(lane == j % LANES, vec, 0))


def _gather_body(n_per_w, bsz, table_hbm, grow_hbm, rstart_hbm, posn_hbm,
                 meta_hbm, out_hbm, grow_v, rstart_v, posn_v, meta_v,
                 buf0, buf1, buf2, buf3, g0, g1, g2, g3, s0, s1, s2, s3):
    wid = lax.axis_index("s") * NC + lax.axis_index("c")
    base = wid * n_per_w
    bufs = (buf0, buf1, buf2, buf3)
    gsems = (g0, g1, g2, g3)
    ssems = (s0, s1, s2, s3)

    def gwait(b):
        pltpu.make_async_copy(table_hbm.at[pl.ds(0, 1)], bufs[b],
                              gsems[b]).wait()

    def swait_n(b, n):
        def one(i, carry):
            pltpu.make_async_copy(table_hbm.at[pl.ds(0, 1)], bufs[b],
                                  ssems[b]).wait()
            return carry
        lax.fori_loop(0, n, one, 0)

    def gather(r, b):
        row = _extract(grow_v, r)
        pltpu.async_copy(table_hbm.at[pl.ds(row, 1)], bufs[b], gsems[b])

    pltpu.sync_copy(grow_hbm.at[wid], grow_v)
    pltpu.sync_copy(rstart_hbm.at[wid], rstart_v)
    pltpu.sync_copy(posn_hbm.at[wid], posn_v)
    pltpu.sync_copy(meta_hbm.at[wid], meta_v)
    nruns = _extract(meta_v, 0)
    nr4 = ((nruns + NBUF - 1) // NBUF) * NBUF

    for r in range(NBUF - 1):
        gather(r, r)

    def step(r, b, carry):
        cnts = list(carry)
        bf = (b + NBUF - 1) % NBUF
        swait_n(bf, cnts[bf])
        gather(r + NBUF - 1, bf)
        gwait(b)
        k0 = _extract(rstart_v, r)
        k1 = _extract(rstart_v, r + 1)

        def one(k, carry2):
            p = base + _extract(posn_v, k)
            pltpu.async_copy(bufs[b],
                             out_hbm.at[p // bsz, pl.ds(p % bsz, 1)],
                             ssems[b])
            return carry2
        lax.fori_loop(k0, k1, one, 0)

        cnts[bf] = 0
        cnts[b] = k1 - k0
        return tuple(cnts)

    def ring_body(rp, carry):
        for b in range(NBUF):
            carry = step(rp + b, b, carry)
        return carry

    carry_out = pl.loop(0, nr4, step=NBUF,
                        init_carry=(jnp.int32(0),) * NBUF)(ring_body)

    for b in range(NBUF - 1):
        gwait(b)
    for b in range(NBUF):
        swait_n(b, carry_out[b])


def kernel(prefix, embedding_table):
    bsz, toks = prefix.shape
    vocab, dim = embedding_table.shape
    n_lookups = bsz * toks
    n_per_w = n_lookups // NW

    seg = jnp.asarray(prefix, jnp.int32).T.reshape(NW, n_per_w)
    offs = (jnp.arange(NW, dtype=jnp.int32) * vocab) // NW
    key = (seg - offs[:, None]) % vocab
    karange = jnp.arange(n_per_w, dtype=jnp.int32)
    spack = jnp.sort(key * 256 + karange[None, :], axis=1)
    keys_sorted = spack // 256
    order = spack % 256
    srt = (keys_sorted + offs[:, None]) % vocab
    first = jnp.concatenate(
        [jnp.ones((NW, 1), bool),
         keys_sorted[:, 1:] != keys_sorted[:, :-1]], axis=1)
    rid = jnp.cumsum(first, axis=1).astype(jnp.int32) - 1
    nruns = rid[:, -1] + 1

    wrow = jnp.arange(NW)[:, None]
    tgt = jnp.where(first, rid, RUN_PAD)
    grow = jnp.zeros((NW, RUN_PAD + 1), jnp.int32)
    grow = grow.at[wrow, tgt].set(srt)[:, :RUN_PAD]
    run_valid = jnp.arange(RUN_PAD)[None, :] < nruns[:, None]
    grow = jnp.where(run_valid, grow, srt[:, -1:])
    rstart = jnp.full((NW, RUN_PAD + 1), n_per_w, jnp.int32)
    rstart = rstart.at[wrow, tgt].set(karange[None, :])[:, :RUN_PAD]

    n_pos = ((n_per_w + LANES - 1) // LANES) * LANES
    posn = jnp.zeros((NW, n_pos), jnp.int32).at[:, :n_per_w].set(order)
    grow = grow.reshape(NW, RUN_PAD // LANES, LANES)
    rstart = rstart.reshape(NW, RUN_PAD // LANES, LANES)
    posn = posn.reshape(NW, n_pos // LANES, LANES)
    meta = jnp.zeros((NW, LANES), jnp.int32).at[:, 0].set(nruns)
    meta = meta.reshape(NW, 1, LANES)

    mesh = plsc.VectorSubcoreMesh(core_axis_name="c", subcore_axis_name="s")
    sc_gather = pl.kernel(
        functools.partial(_gather_body, n_per_w, bsz),
        out_type=jax.ShapeDtypeStruct((toks, bsz, dim), jnp.float32),
        mesh=mesh,
        compiler_params=pltpu.CompilerParams(use_tc_tiling_on_sc=True,
                                             needs_layout_passes=False),
        scratch_types=(
            [pltpu.VMEM((RUN_PAD // LANES, LANES), jnp.int32),
             pltpu.VMEM((RUN_PAD // LANES, LANES), jnp.int32),
             pltpu.VMEM((n_pos // LANES, LANES), jnp.int32),
             pltpu.VMEM((1, LANES), jnp.int32)]
            + [pltpu.VMEM((1, dim), jnp.float32) for _ in range(NBUF)]
            + [pltpu.SemaphoreType.DMA for _ in range(2 * NBUF)]
        ),
    )
    out = sc_gather(embedding_table, grow, rstart, posn, meta)
    return jnp.swapaxes(out, 0, 1)


# speedup vs baseline: 3.9548x; 1.1113x over previous
import functools

import jax
import jax.numpy as jnp
from jax import lax
from jax.experimental import pallas as pl
from jax.experimental.pallas import tpu as pltpu
from jax.experimental.pallas import tpu_sc as plsc

NC = 2
NS = 16
NW = NC * NS
NBUF = 4
LANES = 16
RUN_PAD = 224


def _extract(ref, j):
    vec = ref[j // LANES, :]
    lane = lax.iota(jnp.int32, LANES)
    return jnp.max(jnp.where(lane == j % LANES, vec, 0))


def _gather_body(n_per_w, bsz, table_hbm, grow_hbm, rstart_hbm, posn_hbm,
                 meta_hbm, out_hbm, grow_v, rstart_v, posn_v, meta_v,
                 buf0, buf1, buf2, buf3, g0, g1, g2, g3, s0, s1, s2, s3):
    wid = lax.axis_index("s") * NC + lax.axis_index("c")
    base = wid * n_per_w
    bufs = (buf0, buf1, buf2, buf3)
    gsems = (g0, g1, g2, g3)
    ssems = (s0, s1, s2, s3)

    def gwait(b):
        pltpu.make_async_copy(table_hbm.at[pl.ds(0, 1)], bufs[b],
                              gsems[b]).wait()

    def swait_n(b, n):
        def one(i, carry):
            pltpu.make_async_copy(table_hbm.at[pl.ds(0, 1)], bufs[b],
                                  ssems[b]).wait()
            return carry
        lax.fori_loop(0, n, one, 0)

    def gather(r, b):
        row = _extract(grow_v, r)
        pltpu.async_copy(table_hbm.at[pl.ds(row, 1)], bufs[b], gsems[b])

    pltpu.sync_copy(grow_hbm.at[wid], grow_v)
    pltpu.sync_copy(rstart_hbm.at[wid], rstart_v)
    pltpu.sync_copy(posn_hbm.at[wid], posn_v)
    pltpu.sync_copy(meta_hbm.at[wid], meta_v)
    nruns = _extract(meta_v, 0)
    nr4 = ((nruns + NBUF - 1) // NBUF) * NBUF

    for r in range(NBUF - 1):
        gather(r, r)

    def step(r, b, carry):
        cnts = list(carry)
        bf = (b + NBUF - 1) % NBUF
        swait_n(bf, cnts[bf])
        gather(r + NBUF - 1, bf)
        gwait(b)
        k0 = _extract(rstart_v, r)
        k1 = _extract(rstart_v, r + 1)

        def one(k, carry2):
            p = base + _extract(posn_v, k)
            pltpu.async_copy(bufs[b],
                             out_hbm.at[p // bsz, pl.ds(p % bsz, 1)],
                             ssems[b])
            return carry2
        lax.fori_loop(k0, k1, one, 0)

        cnts[bf] = 0
        cnts[b] = k1 - k0
        return tuple(cnts)

    def ring_body(rp, carry):
        for b in range(NBUF):
            carry = step(rp + b, b, carry)
        return carry

    carry_out = pl.loop(0, nr4, step=NBUF,
                        init_carry=(jnp.int32(0),) * NBUF)(ring_body)

    for b in range(NBUF - 1):
        gwait(b)
    for b in range(NBUF):
        swait_n(b, carry_out[b])


def kernel(prefix, embedding_table):
    bsz, toks = prefix.shape
    vocab, dim = embedding_table.shape
    n_lookups = bsz * toks
    n_per_w = n_lookups // NW

    seg = jnp.asarray(prefix, jnp.int32).T.reshape(NW, n_per_w)
    offs = (jnp.arange(NW, dtype=jnp.int32) * vocab) // NW
    key = (seg - offs[:, None]) % vocab
    karange = jnp.arange(n_per_w, dtype=jnp.int32)
    spack = jnp.sort(key * 256 + karange[None, :], axis=1)
    keys_sorted = spack // 256
    order = spack % 256
    srt = (keys_sorted + offs[:, None]) % vocab
    first = jnp.concatenate(
        [jnp.ones((NW, 1), bool),
         keys_sorted[:, 1:] != keys_sorted[:, :-1]], axis=1)
    rid = jnp.cumsum(first, axis=1).astype(jnp.int32) - 1
    nruns = rid[:, -1] + 1

    rec = jnp.where(first, (rid << 16) | (srt << 8) | karange[None, :],
                    jnp.int32(1) << 30)
    srec = jnp.sort(rec, axis=1)
    pad_cols = RUN_PAD - n_per_w
    grow = jnp.concatenate(
        [(srec >> 8) & 255, jnp.zeros((NW, pad_cols), jnp.int32)], axis=1)
    rstart = jnp.concatenate(
        [srec & 255, jnp.full((NW, pad_cols), n_per_w, jnp.int32)], axis=1)
    run_valid = jnp.arange(RUN_PAD)[None, :] < nruns[:, None]
    grow = jnp.where(run_valid, grow, srt[:, -1:])
    rstart = jnp.where(run_valid, rstart, n_per_w)

    n_pos = ((n_per_w + LANES - 1) // LANES) * LANES
    posn = jnp.zeros((NW, n_pos), jnp.int32).at[:, :n_per_w].set(order)
    grow = grow.reshape(NW, RUN_PAD // LANES, LANES)
    rstart = rstart.reshape(NW, RUN_PAD // LANES, LANES)
    posn = posn.reshape(NW, n_pos // LANES, LANES)
    meta = jnp.zeros((NW, LANES), jnp.int32).at[:, 0].set(nruns)
    meta = meta.reshape(NW, 1, LANES)

    mesh = plsc.VectorSubcoreMesh(core_axis_name="c", subcore_axis_name="s")
    sc_gather = pl.kernel(
        functools.partial(_gather_body, n_per_w, bsz),
        out_type=jax.ShapeDtypeStruct((toks, bsz, dim), jnp.float32),
        mesh=mesh,
        compiler_params=pltpu.CompilerParams(use_tc_tiling_on_sc=True,
                                             needs_layout_passes=False),
        scratch_types=(
            [pltpu.VMEM((RUN_PAD // LANES, LANES), jnp.int32),
             pltpu.VMEM((RUN_PAD // LANES, LANES), jnp.int32),
             pltpu.VMEM((n_pos // LANES, LANES), jnp.int32),
             pltpu.VMEM((1, LANES), jnp.int32)]
            + [pltpu.VMEM((1, dim), jnp.float32) for _ in range(NBUF)]
            + [pltpu.SemaphoreType.DMA for _ in range(2 * NBUF)]
        ),
    )
    out = sc_gather(embedding_table, grow, rstart, posn, meta)
    return jnp.swapaxes(out, 0, 1)
